# baseline (speedup 1.0000x reference)
import jax
import jax.numpy as jnp
from jax.experimental import pallas as pl

N_PAPER = 10000
N_AUTHOR = 10000


def _mean_agg(x_src, edge_index, n_dst):
    src = edge_index[0]
    dst = edge_index[1]
    msgs = jnp.take(x_src, src, axis=0)
    s = jax.ops.segment_sum(msgs, dst, num_segments=n_dst)
    cnt = jax.ops.segment_sum(jnp.ones((edge_index.shape[1],), x_src.dtype), dst, num_segments=n_dst)
    return s / jnp.clip(cnt, 1.0)[:, None]


def _relu_pallas(x):
    def body(x_ref, o_ref):
        o_ref[...] = jnp.maximum(x_ref[...], 0.0)
    return pl.pallas_call(
        body, out_shape=jax.ShapeDtypeStruct(x.shape, x.dtype))(x)


def kernel(x_paper, emb_author, Wr1_cites, Wr1_writes, Wr1_written, Wroot1_paper, broot1_paper, Wroot1_author, broot1_author, Wr2_cites, Wr2_writes, Wr2_written, Wroot2_paper, broot2_paper, Wroot2_author, broot2_author, edge_index_cites, edge_index_writes, edge_index_written):
    def conv(xp, xa, Wrc, Wrw, Wrn, Wrp, brp, Wra, bra):
        op = xp @ Wrp + brp
        oa = xa @ Wra + bra
        op = op + _mean_agg(xp, edge_index_cites, N_PAPER) @ Wrc
        op = op + _mean_agg(xa, edge_index_writes, N_PAPER) @ Wrw
        oa = oa + _mean_agg(xp, edge_index_written, N_AUTHOR) @ Wrn
        return op, oa

    p, a = conv(x_paper, emb_author, Wr1_cites, Wr1_writes, Wr1_written, Wroot1_paper, broot1_paper, Wroot1_author, broot1_author)
    p = _relu_pallas(p)
    a = _relu_pallas(a)
    p, a = conv(p, a, Wr2_cites, Wr2_writes, Wr2_written, Wroot2_paper, broot2_paper, Wroot2_author, broot2_author)
    return p, a


# speedup vs baseline: 2.6689x; 2.6689x over previous
import jax
import jax.numpy as jnp
from jax import lax
from jax.experimental import pallas as pl
from jax.experimental.pallas import tpu as pltpu
from jax.experimental.pallas import tpu_sc as plsc

N = 10000
N_PAD = 10240
DUMMY = 10000
E = 160000
E_PAD = 163840
D_IN = 256
D_H = 256
D_OUT = 128

RB = 512
NB = N_PAD // RB
NTILES = 16
STRIPE = N_PAD // NTILES
CH1 = (E_PAD // NTILES) // 128
CH2 = (E_PAD // (2 * NTILES)) // 128
IDXG = 8

_mesh = plsc.VectorSubcoreMesh(core_axis_name="c", subcore_axis_name="s")


def _run_chunks(tab, src_hbm, dst_hbm, row_base, ngroups, acc,
                src_v, dst_v, bufA, bufB, sems, off=None):
    semA, semA2, semB, semB2, semSA, semSB = sems
    NSUB = 2
    SUBR = 128 // NSUB

    def fire(j, buf, s1, s2):
        return tuple(
            pltpu.async_copy(tab.at[src_v.at[j, pl.ds(k * SUBR, SUBR)]],
                             buf.at[pl.ds(k * SUBR, SUBR)],
                             s1 if k % 2 == 0 else s2)
            for k in range(NSUB))

    def group(g, carry):
        base = row_base + g * IDXG
        pltpu.sync_copy(src_hbm.at[pl.ds(base, IDXG)], src_v)
        pltpu.sync_copy(dst_hbm.at[pl.ds(base, IDXG)], dst_v)
        if off is not None:
            for k in range(IDXG):
                for q in range(8):
                    src_v[k, pl.ds(q * 16, 16)] = (
                        src_v[k, pl.ds(q * 16, 16)] + off)
        gd = {0: fire(0, bufA, semA, semA2)}
        sd = {}
        for j in range(IDXG):
            buf, ssem = (bufA, semSA) if j % 2 == 0 else (bufB, semSB)
            if j + 1 < IDXG:
                nbuf = bufB if j % 2 == 0 else bufA
                ns1, ns2 = (semB, semB2) if j % 2 == 0 else (semA, semA2)
                if j >= 1:
                    sd[j - 1].wait()
                gd[j + 1] = fire(j + 1, nbuf, ns1, ns2)
            for d in gd[j]:
                d.wait()
            sd[j] = pltpu.async_copy(buf, acc.at[dst_v.at[j]], ssem, add=True)
        sd[IDXG - 2].wait()
        sd[IDXG - 1].wait()
        return carry
    lax.fori_loop(0, ngroups, group, 0)


def _matmul_split(x, Wcat, bcat):
    K = x.shape[1]
    S = Wcat.shape[1] // 128
    b2 = bcat.reshape(S, 1, 128)

    def body(x_ref, w_ref, b_ref, o_ref):
        o_ref[0] = jnp.dot(x_ref[...], w_ref[...],
                           preferred_element_type=jnp.float32) + b_ref[0]

    return pl.pallas_call(
        body,
        grid=(NB, S),
        in_specs=[
            pl.BlockSpec((RB, K), lambda i, j: (i, 0)),
            pl.BlockSpec((K, 128), lambda i, j: (0, j)),
            pl.BlockSpec((1, 1, 128), lambda i, j: (j, 0, 0)),
        ],
        out_specs=pl.BlockSpec((1, RB, 128), lambda i, j: (j, i, 0)),
        out_shape=jax.ShapeDtypeStruct((S, N_PAD, 128), jnp.float32),
    )(x, Wcat, b2)


def _inv(cnt_ref):
    return 1.0 / jnp.maximum(cnt_ref[0] + cnt_ref[1], 1.0)


def _epilogue1_paper(HP1, S1c, S1w, cnt_c, cnt_w):
    def body(hp_ref, sc_ref, sw_ref, cc_ref, cw_ref, o_ref):
        o_ref[...] = jnp.maximum(
            hp_ref[0] + sc_ref[0] * _inv(cc_ref) + sw_ref[0] * _inv(cw_ref), 0.0)

    return pl.pallas_call(
        body,
        grid=(NB, 2),
        in_specs=[
            pl.BlockSpec((1, RB, 128), lambda i, h: (h, i, 0)),
            pl.BlockSpec((1, RB, 128), lambda i, h: (h, i, 0)),
            pl.BlockSpec((1, RB, 128), lambda i, h: (h, i, 0)),
            pl.BlockSpec((2, RB, 1), lambda i, h: (0, i, 0)),
            pl.BlockSpec((2, RB, 1), lambda i, h: (0, i, 0)),
        ],
        out_specs=pl.BlockSpec((RB, 128), lambda i, h: (i, h)),
        out_shape=jax.ShapeDtypeStruct((N_PAD, 256), jnp.float32),
    )(HP1, S1c, S1w, cnt_c, cnt_w)


def _epilogue1_author(HA1, S1n, cnt_n):
    def body(ha_ref, sn_ref, cn_ref, o_ref):
        o_ref[...] = jnp.maximum(ha_ref[0] + sn_ref[0] * _inv(cn_ref), 0.0)

    return pl.pallas_call(
        body,
        grid=(NB, 2),
        in_specs=[
            pl.BlockSpec((1, RB, 128), lambda i, h: (h, i, 0)),
            pl.BlockSpec((1, RB, 128), lambda i, h: (h, i, 0)),
            pl.BlockSpec((2, RB, 1), lambda i, h: (0, i, 0)),
        ],
        out_specs=pl.BlockSpec((RB, 128), lambda i, h: (i, h)),
        out_shape=jax.ShapeDtypeStruct((N_PAD, 256), jnp.float32),
    )(HA1, S1n, cnt_n)


def _epilogue2_paper(HP2, P2c, P2w, cnt_c, cnt_w):
    def body(hp_ref, pc_ref, pw_ref, cc_ref, cw_ref, o_ref):
        o_ref[...] = (hp_ref[0]
                      + (pc_ref[0] + pc_ref[1]) * _inv(cc_ref)
                      + (pw_ref[0] + pw_ref[1]) * _inv(cw_ref))

    return pl.pallas_call(
        body,
        grid=(NB,),
        in_specs=[
            pl.BlockSpec((1, RB, 128), lambda i: (0, i, 0)),
            pl.BlockSpec((2, RB, 128), lambda i: (0, i, 0)),
            pl.BlockSpec((2, RB, 128), lambda i: (0, i, 0)),
            pl.BlockSpec((2, RB, 1), lambda i: (0, i, 0)),
            pl.BlockSpec((2, RB, 1), lambda i: (0, i, 0)),
        ],
        out_specs=pl.BlockSpec((RB, 128), lambda i: (i, 0)),
        out_shape=jax.ShapeDtypeStruct((N_PAD, 128), jnp.float32),
    )(HP2, P2c, P2w, cnt_c, cnt_w)


def _epilogue2_author(HA2, P2n, cnt_n):
    def body(ha_ref, pn_ref, cn_ref, o_ref):
        o_ref[...] = ha_ref[0] + (pn_ref[0] + pn_ref[1]) * _inv(cn_ref)

    return pl.pallas_call(
        body,
        grid=(NB,),
        in_specs=[
            pl.BlockSpec((1, RB, 128), lambda i: (0, i, 0)),
            pl.BlockSpec((2, RB, 128), lambda i: (0, i, 0)),
            pl.BlockSpec((2, RB, 1), lambda i: (0, i, 0)),
        ],
        out_specs=pl.BlockSpec((RB, 128), lambda i: (i, 0)),
        out_shape=jax.ShapeDtypeStruct((N_PAD, 128), jnp.float32),
    )(HA2, P2n, cnt_n)


def _sc_pass1_body(HP1, HA1, srcC, dstC, srcW, dstW, srcN, dstN,
                   zerosD,
                   S1c, S1w, S1n,
                   acc, src_v, dst_v, bufA, bufB, *sems):
    c = lax.axis_index("c")
    t = lax.axis_index("s")
    r0 = t * STRIPE

    def relation(src_hbm, dst_hbm, tab_flat, base_slot, out_ref):
        pltpu.sync_copy(zerosD.at[pl.ds(r0, STRIPE)], acc.at[pl.ds(r0, STRIPE)])
        plsc.subcore_barrier()

        _run_chunks(tab_flat, src_hbm, dst_hbm, t * CH1, CH1 // IDXG, acc,
                    src_v, dst_v, bufA, bufB, sems,
                    off=(base_slot + c) * N_PAD)

        plsc.subcore_barrier()

        @pl.when(c == 0)
        def _():
            pltpu.sync_copy(acc.at[pl.ds(r0, STRIPE)],
                            out_ref.at[0].at[pl.ds(r0, STRIPE)])

        @pl.when(c == 1)
        def _():
            pltpu.sync_copy(acc.at[pl.ds(r0, STRIPE)],
                            out_ref.at[1].at[pl.ds(r0, STRIPE)])

        plsc.subcore_barrier()

    relation(srcC, dstC, HP1, 2, S1c)
    relation(srcW, dstW, HA1, 2, S1w)
    relation(srcN, dstN, HP1, 4, S1n)


def _sc_pass1(HP1, HA1, srcC, dstC, srcW, dstW, srcN, dstN, zerosD):
    out_type = tuple(
        jax.ShapeDtypeStruct((2, N_PAD, 128), jnp.float32) for _ in range(3))
    scratch = [
        pltpu.VMEM_SHARED((N_PAD, 128), jnp.float32),
        pltpu.VMEM((IDXG, 128), jnp.int32),
        pltpu.VMEM((IDXG, 128), jnp.int32),
        pltpu.VMEM((128, 128), jnp.float32),
        pltpu.VMEM((128, 128), jnp.float32),
    ] + [pltpu.SemaphoreType.DMA] * 6
    return pl.kernel(_sc_pass1_body, out_type=out_type, mesh=_mesh,
                     scratch_types=scratch)(
        HP1.reshape(6 * N_PAD, 128), HA1.reshape(4 * N_PAD, 128),
        srcC, dstC, srcW, dstW, srcN, dstN, zerosD)


def _sc_counts_body(dstC, dstW, dstN, zerosD,
                    cnt_c, cnt_w, cnt_n,
                    cnt2d, dst_v, ident, shared_cnt):
    c = lax.axis_index("c")
    t = lax.axis_index("s")
    ones_r = jnp.ones((16,), jnp.float32)
    for k in range(8):
        ident[0, pl.ds(k * 16, 16)] = lax.iota(jnp.int32, 16) + k * 16

    def relation(dst_hbm, out_ref):
        pltpu.sync_copy(zerosD.at[pl.ds(t * 8, 8)],
                        shared_cnt.at[pl.ds(t * 8, 8)])

        def zrow(r, carry):
            for q in range(8):
                cnt2d[r, pl.ds(q * 16, 16)] = jnp.zeros((16,), jnp.float32)
            return carry
        lax.fori_loop(0, 128, zrow, 0)

        base = (c * NTILES + t) * CH2
        pltpu.sync_copy(dst_hbm.at[pl.ds(base, CH2)], dst_v)

        def jrow(j, carry):
            for q in range(8):
                ix = dst_v[j, pl.ds(q * 16, 16)]
                plsc.addupdate_scatter(
                    cnt2d, [lax.shift_right_logical(ix, 7),
                            lax.bitwise_and(ix, 127)], ones_r)
            return carry
        lax.fori_loop(0, CH2, jrow, 0)
        plsc.subcore_barrier()
        pltpu.sync_copy(cnt2d, shared_cnt.at[ident.at[0]], add=True)
        plsc.subcore_barrier()

        @pl.when(c == 0)
        def _():
            pltpu.sync_copy(shared_cnt.at[pl.ds(t * 8, 8)],
                            out_ref.at[0].at[pl.ds(t * 8, 8)])

        @pl.when(c == 1)
        def _():
            pltpu.sync_copy(shared_cnt.at[pl.ds(t * 8, 8)],
                            out_ref.at[1].at[pl.ds(t * 8, 8)])

        plsc.subcore_barrier()

    relation(dstC, cnt_c)
    relation(dstW, cnt_w)
    relation(dstN, cnt_n)


def _sc_counts(dstC, dstW, dstN, zerosD):
    out_type = tuple(
        jax.ShapeDtypeStruct((2, 128, 128), jnp.float32) for _ in range(3))
    scratch = [
        pltpu.VMEM((128, 128), jnp.float32),
        pltpu.VMEM((CH2, 128), jnp.int32),
        pltpu.VMEM((1, 128), jnp.int32),
        pltpu.VMEM_SHARED((128, 128), jnp.float32),
    ]
    raw = pl.kernel(
        _sc_counts_body, out_type=out_type, mesh=_mesh,
        compiler_params=pltpu.CompilerParams(needs_layout_passes=False),
        scratch_types=scratch)(dstC, dstW, dstN, zerosD)
    return tuple(r.reshape(2, 128 * 128, 1)[:, :N_PAD] for r in raw)


def _sc_pass2_body(HP2, HA2, srcC, dstC, srcW, dstW, srcN, dstN,
                   zerosD,
                   P2c, P2w, P2n,
                   acc, src_v, dst_v, bufA, bufB, *sems):
    c = lax.axis_index("c")
    t = lax.axis_index("s")
    r0 = t * STRIPE

    def relation(src_hbm, dst_hbm, tab, out_ref):
        pltpu.sync_copy(zerosD.at[pl.ds(r0, STRIPE)], acc.at[pl.ds(r0, STRIPE)])
        plsc.subcore_barrier()

        _run_chunks(tab, src_hbm, dst_hbm, (c * NTILES + t) * CH2,
                    CH2 // IDXG, acc, src_v, dst_v, bufA, bufB, sems)

        plsc.subcore_barrier()

        @pl.when(c == 0)
        def _():
            pltpu.sync_copy(acc.at[pl.ds(r0, STRIPE)],
                            out_ref.at[0].at[pl.ds(r0, STRIPE)])

        @pl.when(c == 1)
        def _():
            pltpu.sync_copy(acc.at[pl.ds(r0, STRIPE)],
                            out_ref.at[1].at[pl.ds(r0, STRIPE)])

        plsc.subcore_barrier()

    relation(srcC, dstC, HP2.at[1], P2c)
    relation(srcW, dstW, HA2.at[1], P2w)
    relation(srcN, dstN, HP2.at[2], P2n)


def _sc_pass2(HP2, HA2, srcC, dstC, srcW, dstW, srcN, dstN, zerosD):
    out_type = tuple(
        jax.ShapeDtypeStruct((2, N_PAD, 128), jnp.float32) for _ in range(3))
    scratch = [
        pltpu.VMEM_SHARED((N_PAD, 128), jnp.float32),
        pltpu.VMEM((IDXG, 128), jnp.int32),
        pltpu.VMEM((IDXG, 128), jnp.int32),
        pltpu.VMEM((128, 128), jnp.float32),
        pltpu.VMEM((128, 128), jnp.float32),
    ] + [pltpu.SemaphoreType.DMA] * 6
    return pl.kernel(_sc_pass2_body, out_type=out_type, mesh=_mesh,
                     scratch_types=scratch)(
        HP2, HA2, srcC, dstC, srcW, dstW, srcN, dstN, zerosD)


def _pad_edges(edge_index):
    src = jnp.concatenate(
        [edge_index[0], jnp.full((E_PAD - E,), DUMMY, jnp.int32)])
    dst = jnp.concatenate(
        [edge_index[1], jnp.full((E_PAD - E,), DUMMY, jnp.int32)])
    return src.reshape(E_PAD // 128, 128), dst.reshape(E_PAD // 128, 128)


def kernel(x_paper, emb_author, Wr1_cites, Wr1_writes, Wr1_written, Wroot1_paper, broot1_paper, Wroot1_author, broot1_author, Wr2_cites, Wr2_writes, Wr2_written, Wroot2_paper, broot2_paper, Wroot2_author, broot2_author, edge_index_cites, edge_index_writes, edge_index_written):
    f32 = jnp.float32
    pad = ((0, N_PAD - N), (0, 0))
    xp = jnp.pad(x_paper, pad)
    xa = jnp.pad(emb_author, pad)

    srcC, dstC = _pad_edges(edge_index_cites)
    srcW, dstW = _pad_edges(edge_index_writes)
    srcN, dstN = _pad_edges(edge_index_written)

    zerosD = jnp.zeros((N_PAD, 128), f32)

    z = jnp.zeros((256,), f32)
    Wp1 = jnp.concatenate([Wroot1_paper, Wr1_cites, Wr1_written], axis=1)
    bp1 = jnp.concatenate([broot1_paper, z, z])
    Wa1 = jnp.concatenate([Wroot1_author, Wr1_writes], axis=1)
    ba1 = jnp.concatenate([broot1_author, z])

    HP1 = _matmul_split(xp, Wp1, bp1)
    HA1 = _matmul_split(xa, Wa1, ba1)

    cnt_c, cnt_w, cnt_n = _sc_counts(dstC, dstW, dstN, zerosD)
    S1c, S1w, S1n = _sc_pass1(
        HP1, HA1, srcC, dstC, srcW, dstW, srcN, dstN, zerosD)

    p1 = _epilogue1_paper(HP1, S1c, S1w, cnt_c, cnt_w)
    a1 = _epilogue1_author(HA1, S1n, cnt_n)

    z2 = jnp.zeros((128,), f32)
    Wp2 = jnp.concatenate([Wroot2_paper, Wr2_cites, Wr2_written], axis=1)
    bp2 = jnp.concatenate([broot2_paper, z2, z2])
    Wa2 = jnp.concatenate([Wroot2_author, Wr2_writes], axis=1)
    ba2 = jnp.concatenate([broot2_author, z2])

    HP2 = _matmul_split(p1, Wp2, bp2)
    HA2 = _matmul_split(a1, Wa2, ba2)

    P2c, P2w, P2n = _sc_pass2(
        HP2, HA2, srcC, dstC, srcW, dstW, srcN, dstN, zerosD)

    p2 = _epilogue2_paper(HP2, P2c, P2w, cnt_c, cnt_w)
    a2 = _epilogue2_author(HA2, P2n, cnt_n)

    return p2[:N], a2[:N]
